# baseline (speedup 1.0000x reference)
import functools

import jax
import jax.numpy as jnp
from jax import lax
from jax.experimental import pallas as pl
from jax.experimental.pallas import tpu as pltpu
from jax.experimental.pallas import tpu_sc as plsc

VOCAB = 49408
HIDDEN = 768
MAX_POS = 77
BATCH = 1024
SEQ = 77

NC = 2
NS = 16
NW = NC * NS

BPW = BATCH // NW
PG = 8
NPG = 10
GB = 2
NGB = BPW // GB
NCHUNKS = NPG * NGB
ROWS = GB * PG
TAIL = SEQ - (NPG - 1) * PG
LANES = 16
NVEC = HIDDEN // LANES
NBUF = 2


def _body(table_hbm, idx_hbm, pos_hbm, out_hbm, idx_v, pos_v, buf, stg, gsem, ssem):
    wid = lax.axis_index("s") * NC + lax.axis_index("c")
    b00 = wid * BPW

    pltpu.sync_copy(idx_hbm.at[wid], idx_v)
    pltpu.sync_copy(pos_hbm, pos_v)

    def gather_start(c):
        m = lax.rem(c, NBUF)
        pltpu.async_copy(table_hbm.at[idx_v.at[c]], buf.at[m], gsem)

    def gather_wait(c):
        m = lax.rem(c, NBUF)
        pltpu.make_async_copy(table_hbm.at[idx_v.at[c]], buf.at[m], gsem).wait()

    def _scatters(c, fn):
        m = lax.rem(c, NBUF)
        r = lax.div(c, NGB)
        b0 = b00 + lax.rem(c, NGB) * GB

        @pl.when(r < NPG - 1)
        def _():
            fn(stg.at[m], out_hbm.at[pl.ds(b0, GB), pl.ds(r * PG, PG)])

        @pl.when(r == NPG - 1)
        def _():
            fn(stg.at[m, :, pl.ds(0, TAIL)],
               out_hbm.at[pl.ds(b0, GB), pl.ds((NPG - 1) * PG, TAIL)])

    def scatter_start(c):
        _scatters(c, lambda src, dst: pltpu.async_copy(src, dst, ssem))

    def scatter_wait(c):
        _scatters(c, lambda src, dst: pltpu.make_async_copy(src, dst, ssem).wait())

    gather_start(0)

    def chunk_body(c, _):
        @pl.when(c >= 1)
        def _():
            scatter_wait(c - 1)

        @pl.when(c + 1 < NCHUNKS)
        def _():
            gather_start(c + 1)

        gather_wait(c)
        m = lax.rem(c, NBUF)
        p0 = lax.div(c, NGB) * PG

        def col_body(j, _):
            sl = pl.ds(j * LANES, LANES)
            for si in range(PG):
                pv = pos_v[p0 + si, sl]
                for bi in range(GB):
                    stg[m, bi, si, sl] = buf[m, bi * PG + si, sl] + pv
            return 0

        lax.fori_loop(0, NVEC, col_body, 0)

        scatter_start(c)
        return 0

    lax.fori_loop(0, NCHUNKS, chunk_body, 0)
    scatter_wait(NCHUNKS - 1)


_sc_call = functools.partial(
    pl.kernel,
    out_type=jax.ShapeDtypeStruct((BATCH, SEQ, HIDDEN), jnp.float32),
    mesh=plsc.VectorSubcoreMesh(
        core_axis_name="c", subcore_axis_name="s", num_cores=NC, num_subcores=NS
    ),
    scratch_types=[
        pltpu.VMEM((NCHUNKS, ROWS), jnp.int32),
        pltpu.VMEM((NPG * PG, HIDDEN), jnp.float32),
        pltpu.VMEM((NBUF, ROWS, HIDDEN), jnp.float32),
        pltpu.VMEM((NBUF, GB, PG, HIDDEN), jnp.float32),
        pltpu.SemaphoreType.DMA,
        pltpu.SemaphoreType.DMA,
    ],
)(_body)


@jax.jit
def kernel(input_ids, token_table, pos_table):
    ids = input_ids.astype(jnp.int32)
    ids = jnp.pad(ids, ((0, 0), (0, NPG * PG - SEQ)), mode="edge")
    ids = ids.reshape(NW, NGB, GB, NPG, PG).transpose(0, 3, 1, 2, 4)
    ids = ids.reshape(NW, NCHUNKS, ROWS)
    pos = jnp.pad(pos_table, ((0, NPG * PG - SEQ), (0, 0)))
    return _sc_call(token_table, ids, pos)


# speedup vs baseline: 1.7084x; 1.7084x over previous
import functools

import jax
import jax.numpy as jnp
from jax import lax
from jax.experimental import pallas as pl
from jax.experimental.pallas import tpu as pltpu
from jax.experimental.pallas import tpu_sc as plsc

VOCAB = 49408
HIDDEN = 768
MAX_POS = 77
BATCH = 1024
SEQ = 77
SEQP = 80

NC = 2
NS = 16
NW = NC * NS

B = BATCH * SEQ
RPW = B // NW
CHUNK = 32
NCHUNKS = RPW // CHUNK
NBUF = 4


def _sc_body(table_hbm, idx_hbm, oidx_hbm, out_hbm, idx_v, oidx_v, buf, gsem, ssem):
    wid = lax.axis_index("s") * NC + lax.axis_index("c")

    pltpu.sync_copy(idx_hbm.at[wid], idx_v)
    pltpu.sync_copy(oidx_hbm.at[wid], oidx_v)

    def gather_start(c):
        m = lax.rem(c, NBUF)
        pltpu.async_copy(table_hbm.at[idx_v.at[c]], buf.at[m], gsem)

    def gather_wait(c):
        m = lax.rem(c, NBUF)
        pltpu.make_async_copy(table_hbm.at[idx_v.at[c]], buf.at[m], gsem).wait()

    def scatter_start(c):
        m = lax.rem(c, NBUF)
        pltpu.async_copy(buf.at[m], out_hbm.at[oidx_v.at[c]], ssem)

    def scatter_wait(c):
        m = lax.rem(c, NBUF)
        pltpu.make_async_copy(buf.at[m], out_hbm.at[oidx_v.at[c]], ssem).wait()

    for c in range(NBUF - 1):
        gather_start(c)

    def chunk_body(c, _):
        @pl.when(c >= 1)
        def _():
            scatter_wait(c - 1)

        @pl.when(c + NBUF - 1 < NCHUNKS)
        def _():
            gather_start(c + NBUF - 1)

        gather_wait(c)
        scatter_start(c)
        return 0

    lax.fori_loop(0, NCHUNKS, chunk_body, 0)
    scatter_wait(NCHUNKS - 1)


_sc_gather = functools.partial(
    pl.kernel,
    out_type=jax.ShapeDtypeStruct((BATCH * SEQP, HIDDEN), jnp.float32),
    mesh=plsc.VectorSubcoreMesh(
        core_axis_name="c", subcore_axis_name="s", num_cores=NC, num_subcores=NS
    ),
    scratch_types=[
        pltpu.VMEM((NCHUNKS, CHUNK), jnp.int32),
        pltpu.VMEM((NCHUNKS, CHUNK), jnp.int32),
        pltpu.VMEM((NBUF, CHUNK, HIDDEN), jnp.float32),
        pltpu.SemaphoreType.DMA,
        pltpu.SemaphoreType.DMA,
    ],
)(_sc_body)


BB = 8


def _tc_body(rows_ref, pos_ref, out_ref):
    out_ref[...] = rows_ref[:, :SEQ, :] + pos_ref[...][None, :, :]


_tc_add = pl.pallas_call(
    _tc_body,
    out_shape=jax.ShapeDtypeStruct((BATCH, SEQ, HIDDEN), jnp.float32),
    grid=(BATCH // BB,),
    in_specs=[
        pl.BlockSpec((BB, SEQP, HIDDEN), lambda b: (b, 0, 0)),
        pl.BlockSpec((SEQ, HIDDEN), lambda b: (0, 0)),
    ],
    out_specs=pl.BlockSpec((BB, SEQ, HIDDEN), lambda b: (b, 0, 0)),
)


@jax.jit
def kernel(input_ids, token_table, pos_table):
    ids = input_ids.astype(jnp.int32).reshape(NW, NCHUNKS, CHUNK)
    g = jnp.arange(B, dtype=jnp.int32)
    oidx = (g + 3 * (g // SEQ)).reshape(NW, NCHUNKS, CHUNK)
    rows = _sc_gather(token_table, ids, oidx)
    return _tc_add(rows.reshape(BATCH, SEQP, HIDDEN), pos_table)


# speedup vs baseline: 1.7630x; 1.0319x over previous
import functools

import jax
import jax.numpy as jnp
from jax import lax
from jax.experimental import pallas as pl
from jax.experimental.pallas import tpu as pltpu
from jax.experimental.pallas import tpu_sc as plsc

VOCAB = 49408
HIDDEN = 768
MAX_POS = 77
BATCH = 1024
SEQ = 77
SEQP = 80

NC = 2
NS = 16
NW = NC * NS

B = BATCH * SEQ
RPW = B // NW
CHUNK = 32
NCHUNKS = RPW // CHUNK
NBUF = 4


def _sc_body(table_hbm, idx_hbm, oidx_hbm, out_hbm, idx_v, oidx_v, buf, gsem, ssem):
    wid = lax.axis_index("s") * NC + lax.axis_index("c")

    pltpu.sync_copy(idx_hbm.at[wid], idx_v)
    pltpu.sync_copy(oidx_hbm.at[wid], oidx_v)

    def gather_start(c):
        m = lax.rem(c, NBUF)
        pltpu.async_copy(table_hbm.at[idx_v.at[c]], buf.at[m], gsem)

    def gather_wait(c):
        m = lax.rem(c, NBUF)
        pltpu.make_async_copy(table_hbm.at[idx_v.at[c]], buf.at[m], gsem).wait()

    def scatter_start(c):
        m = lax.rem(c, NBUF)
        pltpu.async_copy(buf.at[m], out_hbm.at[oidx_v.at[c]], ssem)

    def scatter_wait(c):
        m = lax.rem(c, NBUF)
        pltpu.make_async_copy(buf.at[m], out_hbm.at[oidx_v.at[c]], ssem).wait()

    for c in range(NBUF - 1):
        gather_start(c)

    def chunk_body(c, _):
        @pl.when(c >= 1)
        def _():
            scatter_wait(c - 1)

        @pl.when(c + NBUF - 1 < NCHUNKS)
        def _():
            gather_start(c + NBUF - 1)

        gather_wait(c)
        scatter_start(c)
        return 0

    lax.fori_loop(0, NCHUNKS, chunk_body, 0)
    scatter_wait(NCHUNKS - 1)


_sc_gather = functools.partial(
    pl.kernel,
    out_type=jax.ShapeDtypeStruct((BATCH * SEQP, HIDDEN), jnp.float32),
    mesh=plsc.VectorSubcoreMesh(
        core_axis_name="c", subcore_axis_name="s", num_cores=NC, num_subcores=NS
    ),
    scratch_types=[
        pltpu.VMEM((NCHUNKS, CHUNK), jnp.int32),
        pltpu.VMEM((NCHUNKS, CHUNK), jnp.int32),
        pltpu.VMEM((NBUF, CHUNK, HIDDEN), jnp.float32),
        pltpu.SemaphoreType.DMA,
        pltpu.SemaphoreType.DMA,
    ],
)(_sc_body)


BB = 32


def _tc_body(rows_ref, pos_ref, out_ref):
    out_ref[...] = rows_ref[:, :SEQ, :] + pos_ref[...][None, :, :]


_tc_add = pl.pallas_call(
    _tc_body,
    out_shape=jax.ShapeDtypeStruct((BATCH, SEQ, HIDDEN), jnp.float32),
    grid=(BATCH // BB,),
    in_specs=[
        pl.BlockSpec((BB, SEQP, HIDDEN), lambda b: (b, 0, 0)),
        pl.BlockSpec((SEQ, HIDDEN), lambda b: (0, 0)),
    ],
    out_specs=pl.BlockSpec((BB, SEQ, HIDDEN), lambda b: (b, 0, 0)),
)


@jax.jit
def kernel(input_ids, token_table, pos_table):
    ids = input_ids.astype(jnp.int32).reshape(NW, NCHUNKS, CHUNK)
    g = jnp.arange(B, dtype=jnp.int32)
    oidx = (g + 3 * (g // SEQ)).reshape(NW, NCHUNKS, CHUNK)
    rows = _sc_gather(token_table, ids, oidx)
    return _tc_add(rows.reshape(BATCH, SEQP, HIDDEN), pos_table)


# speedup vs baseline: 3.5025x; 1.9867x over previous
import functools

import jax
import jax.numpy as jnp
from jax import lax
from jax.experimental import pallas as pl
from jax.experimental.pallas import tpu as pltpu
from jax.experimental.pallas import tpu_sc as plsc

VOCAB = 49408
HIDDEN = 768
MAX_POS = 77
BATCH = 1024
SEQ = 77

NC = 2
NS = 16
NW = NC * NS

B = BATCH * SEQ
BPW = BATCH // NW
LANES = 16
NVEC = HIDDEN // LANES
NBUF = 2


def _body(table_hbm, idx_hbm, pos_hbm, out_hbm, idx_v, pos_v, buf, gsem, ssem):
    wid = lax.axis_index("s") * NC + lax.axis_index("c")
    col0 = wid * BPW

    pltpu.sync_copy(idx_hbm.at[wid], idx_v)
    pltpu.sync_copy(pos_hbm, pos_v)

    def gather_start(p):
        m = lax.rem(p, NBUF)
        pltpu.async_copy(table_hbm.at[idx_v.at[p]], buf.at[m], gsem)

    def gather_wait(p):
        m = lax.rem(p, NBUF)
        pltpu.make_async_copy(table_hbm.at[idx_v.at[p]], buf.at[m], gsem).wait()

    def scatter_start(p):
        m = lax.rem(p, NBUF)
        pltpu.async_copy(buf.at[m], out_hbm.at[pl.ds(p * BATCH + col0, BPW)], ssem)

    def scatter_wait(p):
        m = lax.rem(p, NBUF)
        pltpu.make_async_copy(
            buf.at[m], out_hbm.at[pl.ds(p * BATCH + col0, BPW)], ssem
        ).wait()

    gather_start(0)

    def chunk_body(p, _):
        @pl.when(p >= 1)
        def _():
            scatter_wait(p - 1)

        @pl.when(p + 1 < SEQ)
        def _():
            gather_start(p + 1)

        gather_wait(p)
        m = lax.rem(p, NBUF)

        def col_body(j, _):
            sl = pl.ds(j * LANES, LANES)
            pv = pos_v[p, sl]
            for b in range(BPW):
                buf[m, b, sl] += pv
            return 0

        lax.fori_loop(0, NVEC, col_body, 0)

        scatter_start(p)
        return 0

    lax.fori_loop(0, SEQ, chunk_body, 0)
    scatter_wait(SEQ - 1)


_sc_call = functools.partial(
    pl.kernel,
    out_type=jax.ShapeDtypeStruct((B, HIDDEN), jnp.float32),
    mesh=plsc.VectorSubcoreMesh(
        core_axis_name="c", subcore_axis_name="s", num_cores=NC, num_subcores=NS
    ),
    scratch_types=[
        pltpu.VMEM((SEQ, BPW), jnp.int32),
        pltpu.VMEM((MAX_POS, HIDDEN), jnp.float32),
        pltpu.VMEM((NBUF, BPW, HIDDEN), jnp.float32),
        pltpu.SemaphoreType.DMA,
        pltpu.SemaphoreType.DMA,
    ],
)(_body)


@jax.jit
def kernel(input_ids, token_table, pos_table):
    ids = input_ids.astype(jnp.int32).reshape(NW, BPW, SEQ).transpose(0, 2, 1)
    out = _sc_call(token_table, ids, pos_table)
    return out.reshape(SEQ, BATCH, HIDDEN).transpose(1, 0, 2)
